# baseline (speedup 1.0000x reference)
import functools

import jax
import jax.numpy as jnp
from jax import lax
from jax.experimental import pallas as pl
from jax.experimental.pallas import tpu as pltpu
from jax.experimental.pallas import tpu_sc as plsc

_DIM = 2048
_E = 16
_K = 2
_NTOK = 16384
_NFLAT = _NTOK * _K

_BM = 1024
_NSUB = 16
_TOK_PER_TILE = _NTOK // _NSUB
_ROWS = 16


def _gate_body(x_ref, w_ref, b_ref, s1_ref, s2_ref, e1_ref, e2_ref):
    x = x_ref[...]
    w = w_ref[...]
    b = b_ref[...]
    z = jnp.dot(x, w, preferred_element_type=jnp.float32) + b
    m = jnp.max(z, axis=1, keepdims=True)
    ez = jnp.exp(z - m)
    p = ez / jnp.sum(ez, axis=1, keepdims=True)
    a1 = jnp.argmax(p, axis=1).astype(jnp.int32)
    m1 = jnp.max(p, axis=1)
    cols = lax.broadcasted_iota(jnp.int32, p.shape, 1)
    p2 = jnp.where(cols == a1[:, None], -1.0, p)
    a2 = jnp.argmax(p2, axis=1).astype(jnp.int32)
    m2 = jnp.max(p2, axis=1)
    d = m1 + m2
    s1_ref[...] = m1 / d
    s2_ref[...] = m2 / d
    e1_ref[...] = a1
    e2_ref[...] = a2


def _gate(x, W, b2d):
    grid = (_NTOK // _BM,)
    return pl.pallas_call(
        _gate_body,
        grid=grid,
        in_specs=[
            pl.BlockSpec((_BM, _DIM), lambda i: (i, 0)),
            pl.BlockSpec((_DIM, _E), lambda i: (0, 0)),
            pl.BlockSpec((1, _E), lambda i: (0, 0)),
        ],
        out_specs=[
            pl.BlockSpec((_BM,), lambda i: (i,)),
            pl.BlockSpec((_BM,), lambda i: (i,)),
            pl.BlockSpec((_BM,), lambda i: (i,)),
            pl.BlockSpec((_BM,), lambda i: (i,)),
        ],
        out_shape=[
            jax.ShapeDtypeStruct((_NTOK,), jnp.float32),
            jax.ShapeDtypeStruct((_NTOK,), jnp.float32),
            jax.ShapeDtypeStruct((_NTOK,), jnp.int32),
            jax.ShapeDtypeStruct((_NTOK,), jnp.int32),
        ],
    )(x, W, b2d)


def _route_body(
    e1_hbm, e2_hbm, s1_hbm, s2_hbm,
    oscore_hbm, otok_hbm, ocnt_hbm,
    e1_v, e2_v, s1_v, s2_v,
    hist_v, ptr_v, cnt_v, histmat_v,
    pos_v, val_v, tok_v,
    hist_sh, sem,
):
    wid = lax.axis_index("s")
    tbase = wid * _TOK_PER_TILE
    pltpu.sync_copy(e1_hbm.at[pl.ds(tbase, _TOK_PER_TILE)], e1_v)
    pltpu.sync_copy(e2_hbm.at[pl.ds(tbase, _TOK_PER_TILE)], e2_v)
    pltpu.sync_copy(s1_hbm.at[pl.ds(tbase, _TOK_PER_TILE)], s1_v)
    pltpu.sync_copy(s2_hbm.at[pl.ds(tbase, _TOK_PER_TILE)], s2_v)

    hist_v[...] = jnp.zeros((_E,), jnp.int32)

    def _hist_one(ev):
        occ, last = plsc.scan_count(ev)
        plsc.addupdate_scatter(hist_v, [ev], occ, mask=last)

    def _hist_loop(g, c):
        _hist_one(e1_v[pl.ds(g * 16, 16)])
        _hist_one(e2_v[pl.ds(g * 16, 16)])
        return c

    lax.fori_loop(0, _TOK_PER_TILE // 16, _hist_loop, 0)

    pltpu.sync_copy(hist_v, hist_sh.at[wid])
    plsc.subcore_barrier()
    pltpu.sync_copy(hist_sh, histmat_v)

    total = jnp.zeros((_E,), jnp.int32)
    prefix = jnp.zeros((_E,), jnp.int32)
    for w in range(_NSUB):
        h = histmat_v[w]
        total = total + h
        prefix = prefix + h * (jnp.int32(w) < wid).astype(jnp.int32)
    incl = plsc.cumsum(total)
    start = (incl - total) + prefix
    ptr_v[...] = start

    @pl.when(wid == 0)
    def _():
        cnt_v[...] = total
        pltpu.sync_copy(cnt_v, ocnt_hbm)

    iota = lax.iota(jnp.int32, 16)
    half = iota >> 1
    parity = (iota & 1) == 1

    def _row_loop(r, c):
        for h in range(8):
            tok_idx = r * 64 + h * 8 + half
            ea = plsc.load_gather(e1_v, [tok_idx])
            eb = plsc.load_gather(e2_v, [tok_idx])
            ev = jnp.where(parity, eb, ea)
            occ, last = plsc.scan_count(ev)
            base = plsc.load_gather(ptr_v, [ev])
            pos = base + occ - 1
            plsc.store_scatter(ptr_v, [ev], pos + 1, mask=last)
            sa = plsc.load_gather(s1_v, [tok_idx])
            sb = plsc.load_gather(s2_v, [tok_idx])
            sv = jnp.where(parity, sb, sa)
            pos_v[r, pl.ds(h * 16, 16)] = pos
            val_v[r, pl.ds(h * 16, 16)] = sv
            tok_v[r, pl.ds(h * 16, 16)] = tbase + tok_idx
        return c

    lax.fori_loop(0, _ROWS, _row_loop, 0)

    copies = []
    for r in range(_ROWS):
        copies.append(pltpu.async_copy(val_v.at[r], oscore_hbm.at[pos_v.at[r]], sem))
        copies.append(pltpu.async_copy(tok_v.at[r], otok_hbm.at[pos_v.at[r]], sem))
    for c in copies:
        c.wait()


@functools.cache
def _make_route():
    return pl.kernel(
        _route_body,
        out_type=[
            jax.ShapeDtypeStruct((_NFLAT,), jnp.float32),
            jax.ShapeDtypeStruct((_NFLAT,), jnp.int32),
            jax.ShapeDtypeStruct((_E,), jnp.int32),
        ],
        mesh=plsc.VectorSubcoreMesh(
            core_axis_name="c", subcore_axis_name="s", num_cores=1
        ),
        compiler_params=pltpu.CompilerParams(needs_layout_passes=False),
        scratch_types=[
            pltpu.VMEM((_TOK_PER_TILE,), jnp.int32),
            pltpu.VMEM((_TOK_PER_TILE,), jnp.int32),
            pltpu.VMEM((_TOK_PER_TILE,), jnp.float32),
            pltpu.VMEM((_TOK_PER_TILE,), jnp.float32),
            pltpu.VMEM((_E,), jnp.int32),
            pltpu.VMEM((_E,), jnp.int32),
            pltpu.VMEM((_E,), jnp.int32),
            pltpu.VMEM((_NSUB, _E), jnp.int32),
            pltpu.VMEM((_ROWS, 128), jnp.int32),
            pltpu.VMEM((_ROWS, 128), jnp.float32),
            pltpu.VMEM((_ROWS, 128), jnp.int32),
            pltpu.HBM((_NSUB, _E), jnp.int32),
            pltpu.SemaphoreType.DMA,
        ],
    )


def kernel(x, W, b):
    s1, s2, e1, e2 = _gate(x, W, b.reshape(1, _E))
    top_scores_sorted, token_indices, num_tokens_per_expert = _make_route()(
        e1, e2, s1, s2
    )
    return (top_scores_sorted, token_indices, num_tokens_per_expert)


# speedup vs baseline: 1.9878x; 1.9878x over previous
import functools

import jax
import jax.numpy as jnp
from jax import lax
from jax.experimental import pallas as pl
from jax.experimental.pallas import tpu as pltpu
from jax.experimental.pallas import tpu_sc as plsc

_DIM = 2048
_E = 16
_K = 2
_NTOK = 16384
_NFLAT = _NTOK * _K

_BM = 1024
_NSUB = 16
_TOK_PER_TILE = _NTOK // _NSUB
_ROWS = 16


def _gate_body(x_ref, w_ref, b_ref, s1_ref, s2_ref, e1_ref, e2_ref):
    x = x_ref[...]
    w = w_ref[...]
    b = b_ref[...]
    z = jnp.dot(x, w, preferred_element_type=jnp.float32) + b
    m = jnp.max(z, axis=1, keepdims=True)
    ez = jnp.exp(z - m)
    p = ez / jnp.sum(ez, axis=1, keepdims=True)
    a1 = jnp.argmax(p, axis=1).astype(jnp.int32)
    m1 = jnp.max(p, axis=1)
    cols = lax.broadcasted_iota(jnp.int32, p.shape, 1)
    p2 = jnp.where(cols == a1[:, None], -1.0, p)
    a2 = jnp.argmax(p2, axis=1).astype(jnp.int32)
    m2 = jnp.max(p2, axis=1)
    d = m1 + m2
    s1_ref[...] = m1 / d
    s2_ref[...] = m2 / d
    e1_ref[...] = a1
    e2_ref[...] = a2


def _gate(x, W, b2d):
    grid = (_NTOK // _BM,)
    return pl.pallas_call(
        _gate_body,
        grid=grid,
        in_specs=[
            pl.BlockSpec((_BM, _DIM), lambda i: (i, 0)),
            pl.BlockSpec((_DIM, _E), lambda i: (0, 0)),
            pl.BlockSpec((1, _E), lambda i: (0, 0)),
        ],
        out_specs=[
            pl.BlockSpec((_BM,), lambda i: (i,)),
            pl.BlockSpec((_BM,), lambda i: (i,)),
            pl.BlockSpec((_BM,), lambda i: (i,)),
            pl.BlockSpec((_BM,), lambda i: (i,)),
        ],
        out_shape=[
            jax.ShapeDtypeStruct((_NTOK,), jnp.float32),
            jax.ShapeDtypeStruct((_NTOK,), jnp.float32),
            jax.ShapeDtypeStruct((_NTOK,), jnp.int32),
            jax.ShapeDtypeStruct((_NTOK,), jnp.int32),
        ],
    )(x, W, b2d)


def _route_body(
    e1_hbm, e2_hbm, s1_hbm, s2_hbm,
    oscore_hbm, otok_hbm, ocnt_hbm,
    e1_v, e2_v, s1_v, s2_v,
    hist_v, ptr_v, cnt_v, histmat_v,
    pos_v, val_v, tok_v,
    hist_sh, sc_sh, tk_sh, drain_f, drain_i, sem,
):
    wid = lax.axis_index("s")
    tbase = wid * _TOK_PER_TILE
    pltpu.sync_copy(e1_hbm.at[pl.ds(tbase, _TOK_PER_TILE)], e1_v)
    pltpu.sync_copy(e2_hbm.at[pl.ds(tbase, _TOK_PER_TILE)], e2_v)
    pltpu.sync_copy(s1_hbm.at[pl.ds(tbase, _TOK_PER_TILE)], s1_v)
    pltpu.sync_copy(s2_hbm.at[pl.ds(tbase, _TOK_PER_TILE)], s2_v)

    hist_v[...] = jnp.zeros((_E,), jnp.int32)

    def _hist_one(ev):
        occ, last = plsc.scan_count(ev)
        plsc.addupdate_scatter(hist_v, [ev], occ, mask=last)

    def _hist_loop(g, c):
        _hist_one(e1_v[pl.ds(g * 16, 16)])
        _hist_one(e2_v[pl.ds(g * 16, 16)])
        return c

    lax.fori_loop(0, _TOK_PER_TILE // 16, _hist_loop, 0)

    pltpu.sync_copy(hist_v, hist_sh.at[wid])
    plsc.subcore_barrier()
    pltpu.sync_copy(hist_sh, histmat_v)

    total = jnp.zeros((_E,), jnp.int32)
    prefix = jnp.zeros((_E,), jnp.int32)
    for w in range(_NSUB):
        h = histmat_v[w]
        total = total + h
        prefix = prefix + h * (jnp.int32(w) < wid).astype(jnp.int32)
    incl = plsc.cumsum(total)
    start = (incl - total) + prefix
    ptr_v[...] = start

    @pl.when(wid == 0)
    def _():
        cnt_v[...] = total
        pltpu.sync_copy(cnt_v, ocnt_hbm)

    iota = lax.iota(jnp.int32, 16)
    half = iota >> 1
    parity = (iota & 1) == 1

    def _row_loop(r, c):
        for h in range(8):
            tok_idx = r * 64 + h * 8 + half
            ea = plsc.load_gather(e1_v, [tok_idx])
            eb = plsc.load_gather(e2_v, [tok_idx])
            ev = jnp.where(parity, eb, ea)
            occ, last = plsc.scan_count(ev)
            base = plsc.load_gather(ptr_v, [ev])
            pos = base + occ - 1
            plsc.store_scatter(ptr_v, [ev], pos + 1, mask=last)
            sa = plsc.load_gather(s1_v, [tok_idx])
            sb = plsc.load_gather(s2_v, [tok_idx])
            sv = jnp.where(parity, sb, sa)
            pos_v[r, pl.ds(h * 16, 16)] = pos
            val_v[r, pl.ds(h * 16, 16)] = sv
            tok_v[r, pl.ds(h * 16, 16)] = tbase + tok_idx
        return c

    lax.fori_loop(0, _ROWS, _row_loop, 0)

    copies = []
    for r in range(_ROWS):
        copies.append(pltpu.async_copy(val_v.at[r], sc_sh.at[pos_v.at[r]], sem))
        copies.append(pltpu.async_copy(tok_v.at[r], tk_sh.at[pos_v.at[r]], sem))
    for c in copies:
        c.wait()
    plsc.subcore_barrier()
    obase = wid * (2 * _TOK_PER_TILE)
    pltpu.sync_copy(sc_sh.at[pl.ds(obase, 2 * _TOK_PER_TILE)], drain_f)
    pltpu.sync_copy(tk_sh.at[pl.ds(obase, 2 * _TOK_PER_TILE)], drain_i)
    pltpu.sync_copy(drain_f, oscore_hbm.at[pl.ds(obase, 2 * _TOK_PER_TILE)])
    pltpu.sync_copy(drain_i, otok_hbm.at[pl.ds(obase, 2 * _TOK_PER_TILE)])


@functools.cache
def _make_route():
    return pl.kernel(
        _route_body,
        out_type=[
            jax.ShapeDtypeStruct((_NFLAT,), jnp.float32),
            jax.ShapeDtypeStruct((_NFLAT,), jnp.int32),
            jax.ShapeDtypeStruct((_E,), jnp.int32),
        ],
        mesh=plsc.VectorSubcoreMesh(
            core_axis_name="c", subcore_axis_name="s", num_cores=1
        ),
        compiler_params=pltpu.CompilerParams(needs_layout_passes=False),
        scratch_types=[
            pltpu.VMEM((_TOK_PER_TILE,), jnp.int32),
            pltpu.VMEM((_TOK_PER_TILE,), jnp.int32),
            pltpu.VMEM((_TOK_PER_TILE,), jnp.float32),
            pltpu.VMEM((_TOK_PER_TILE,), jnp.float32),
            pltpu.VMEM((_E,), jnp.int32),
            pltpu.VMEM((_E,), jnp.int32),
            pltpu.VMEM((_E,), jnp.int32),
            pltpu.VMEM((_NSUB, _E), jnp.int32),
            pltpu.VMEM((_ROWS, 128), jnp.int32),
            pltpu.VMEM((_ROWS, 128), jnp.float32),
            pltpu.VMEM((_ROWS, 128), jnp.int32),
            pltpu.HBM((_NSUB, _E), jnp.int32),
            pltpu.VMEM_SHARED((_NFLAT,), jnp.float32),
            pltpu.VMEM_SHARED((_NFLAT,), jnp.int32),
            pltpu.VMEM((2 * _TOK_PER_TILE,), jnp.float32),
            pltpu.VMEM((2 * _TOK_PER_TILE,), jnp.int32),
            pltpu.SemaphoreType.DMA,
        ],
    )


def kernel(x, W, b):
    s1, s2, e1, e2 = _gate(x, W, b.reshape(1, _E))
    top_scores_sorted, token_indices, num_tokens_per_expert = _make_route()(
        e1, e2, s1, s2
    )
    return (top_scores_sorted, token_indices, num_tokens_per_expert)


# speedup vs baseline: 2.7315x; 1.3741x over previous
import functools

import jax
import jax.numpy as jnp
from jax import lax
from jax.experimental import pallas as pl
from jax.experimental.pallas import tpu as pltpu
from jax.experimental.pallas import tpu_sc as plsc

_DIM = 2048
_E = 16
_K = 2
_NTOK = 16384
_NFLAT = _NTOK * _K

_BM = 1024
_NSUB = 16
_TOK_PER_TILE = _NTOK // _NSUB
_ROWS = 16


def _gate_body(x_ref, w_ref, b_ref, s1_ref, s2_ref, e1_ref, e2_ref):
    x = x_ref[...]
    w = w_ref[...]
    b = b_ref[...]
    z = jnp.dot(x, w, preferred_element_type=jnp.float32) + b
    zt = z.T
    rows = lax.broadcasted_iota(jnp.int32, zt.shape, 0)
    m1 = jnp.max(zt, axis=0)
    a1 = jnp.min(jnp.where(zt == m1[None, :], rows, _E), axis=0).astype(jnp.int32)
    z2 = jnp.where(rows == a1[None, :], -jnp.inf, zt)
    m2 = jnp.max(z2, axis=0)
    a2 = jnp.min(jnp.where(z2 == m2[None, :], rows, _E), axis=0).astype(jnp.int32)
    t = jnp.exp(m2 - m1)
    s1 = 1.0 / (1.0 + t)
    s1_ref[...] = s1
    s2_ref[...] = t * s1
    e1_ref[...] = a1
    e2_ref[...] = a2


def _gate(x, W, b2d):
    grid = (_NTOK // _BM,)
    return pl.pallas_call(
        _gate_body,
        grid=grid,
        in_specs=[
            pl.BlockSpec((_BM, _DIM), lambda i: (i, 0)),
            pl.BlockSpec((_DIM, _E), lambda i: (0, 0)),
            pl.BlockSpec((1, _E), lambda i: (0, 0)),
        ],
        out_specs=[
            pl.BlockSpec((_BM,), lambda i: (i,)),
            pl.BlockSpec((_BM,), lambda i: (i,)),
            pl.BlockSpec((_BM,), lambda i: (i,)),
            pl.BlockSpec((_BM,), lambda i: (i,)),
        ],
        out_shape=[
            jax.ShapeDtypeStruct((_NTOK,), jnp.float32),
            jax.ShapeDtypeStruct((_NTOK,), jnp.float32),
            jax.ShapeDtypeStruct((_NTOK,), jnp.int32),
            jax.ShapeDtypeStruct((_NTOK,), jnp.int32),
        ],
    )(x, W, b2d)


def _route_body(
    e1_hbm, e2_hbm, s1_hbm, s2_hbm,
    oscore_hbm, otok_hbm, ocnt_hbm,
    e1_v, e2_v, s1_v, s2_v,
    hist_v, ptr_v, cnt_v, histmat_v,
    pos_v, val_v, tok_v,
    hist_sh, sc_sh, tk_sh, drain_f, drain_i, sem,
):
    wid = lax.axis_index("s")
    tbase = wid * _TOK_PER_TILE
    pltpu.sync_copy(e1_hbm.at[pl.ds(tbase, _TOK_PER_TILE)], e1_v)
    pltpu.sync_copy(e2_hbm.at[pl.ds(tbase, _TOK_PER_TILE)], e2_v)
    pltpu.sync_copy(s1_hbm.at[pl.ds(tbase, _TOK_PER_TILE)], s1_v)
    pltpu.sync_copy(s2_hbm.at[pl.ds(tbase, _TOK_PER_TILE)], s2_v)

    hist_v[...] = jnp.zeros((_E,), jnp.int32)

    def _hist_one(ev):
        occ, last = plsc.scan_count(ev)
        plsc.addupdate_scatter(hist_v, [ev], occ, mask=last)

    def _hist_loop(g, c):
        _hist_one(e1_v[pl.ds(g * 16, 16)])
        _hist_one(e2_v[pl.ds(g * 16, 16)])
        return c

    lax.fori_loop(0, _TOK_PER_TILE // 16, _hist_loop, 0)

    pltpu.sync_copy(hist_v, hist_sh.at[wid])
    plsc.subcore_barrier()
    pltpu.sync_copy(hist_sh, histmat_v)

    total = jnp.zeros((_E,), jnp.int32)
    prefix = jnp.zeros((_E,), jnp.int32)
    for w in range(_NSUB):
        h = histmat_v[w]
        total = total + h
        prefix = prefix + h * (jnp.int32(w) < wid).astype(jnp.int32)
    incl = plsc.cumsum(total)
    start = (incl - total) + prefix
    ptr_v[...] = start

    @pl.when(wid == 0)
    def _():
        cnt_v[...] = total
        pltpu.sync_copy(cnt_v, ocnt_hbm)

    iota = lax.iota(jnp.int32, 16)
    half = iota >> 1
    parity = (iota & 1) == 1

    def _row_loop(r, c):
        for h in range(8):
            tok_idx = r * 64 + h * 8 + half
            ea = plsc.load_gather(e1_v, [tok_idx])
            eb = plsc.load_gather(e2_v, [tok_idx])
            ev = jnp.where(parity, eb, ea)
            occ, last = plsc.scan_count(ev)
            base = plsc.load_gather(ptr_v, [ev])
            pos = base + occ - 1
            plsc.store_scatter(ptr_v, [ev], pos + 1, mask=last)
            sa = plsc.load_gather(s1_v, [tok_idx])
            sb = plsc.load_gather(s2_v, [tok_idx])
            sv = jnp.where(parity, sb, sa)
            pos_v[r, pl.ds(h * 16, 16)] = pos
            val_v[r, pl.ds(h * 16, 16)] = sv
            tok_v[r, pl.ds(h * 16, 16)] = tbase + tok_idx
        return c

    lax.fori_loop(0, _ROWS, _row_loop, 0)

    copies = []
    for r in range(_ROWS):
        copies.append(pltpu.async_copy(val_v.at[r], sc_sh.at[pos_v.at[r]], sem))
        copies.append(pltpu.async_copy(tok_v.at[r], tk_sh.at[pos_v.at[r]], sem))
    for c in copies:
        c.wait()
    plsc.subcore_barrier()
    obase = wid * (2 * _TOK_PER_TILE)
    pltpu.sync_copy(sc_sh.at[pl.ds(obase, 2 * _TOK_PER_TILE)], drain_f)
    pltpu.sync_copy(tk_sh.at[pl.ds(obase, 2 * _TOK_PER_TILE)], drain_i)
    pltpu.sync_copy(drain_f, oscore_hbm.at[pl.ds(obase, 2 * _TOK_PER_TILE)])
    pltpu.sync_copy(drain_i, otok_hbm.at[pl.ds(obase, 2 * _TOK_PER_TILE)])


@functools.cache
def _make_route():
    return pl.kernel(
        _route_body,
        out_type=[
            jax.ShapeDtypeStruct((_NFLAT,), jnp.float32),
            jax.ShapeDtypeStruct((_NFLAT,), jnp.int32),
            jax.ShapeDtypeStruct((_E,), jnp.int32),
        ],
        mesh=plsc.VectorSubcoreMesh(
            core_axis_name="c", subcore_axis_name="s", num_cores=1
        ),
        compiler_params=pltpu.CompilerParams(needs_layout_passes=False),
        scratch_types=[
            pltpu.VMEM((_TOK_PER_TILE,), jnp.int32),
            pltpu.VMEM((_TOK_PER_TILE,), jnp.int32),
            pltpu.VMEM((_TOK_PER_TILE,), jnp.float32),
            pltpu.VMEM((_TOK_PER_TILE,), jnp.float32),
            pltpu.VMEM((_E,), jnp.int32),
            pltpu.VMEM((_E,), jnp.int32),
            pltpu.VMEM((_E,), jnp.int32),
            pltpu.VMEM((_NSUB, _E), jnp.int32),
            pltpu.VMEM((_ROWS, 128), jnp.int32),
            pltpu.VMEM((_ROWS, 128), jnp.float32),
            pltpu.VMEM((_ROWS, 128), jnp.int32),
            pltpu.HBM((_NSUB, _E), jnp.int32),
            pltpu.VMEM_SHARED((_NFLAT,), jnp.float32),
            pltpu.VMEM_SHARED((_NFLAT,), jnp.int32),
            pltpu.VMEM((2 * _TOK_PER_TILE,), jnp.float32),
            pltpu.VMEM((2 * _TOK_PER_TILE,), jnp.int32),
            pltpu.SemaphoreType.DMA,
        ],
    )


def kernel(x, W, b):
    s1, s2, e1, e2 = _gate(x, W, b.reshape(1, _E))
    top_scores_sorted, token_indices, num_tokens_per_expert = _make_route()(
        e1, e2, s1, s2
    )
    return (top_scores_sorted, token_indices, num_tokens_per_expert)


# speedup vs baseline: 2.8935x; 1.0593x over previous
import functools

import jax
import jax.numpy as jnp
from jax import lax
from jax.experimental import pallas as pl
from jax.experimental.pallas import tpu as pltpu
from jax.experimental.pallas import tpu_sc as plsc

_DIM = 2048
_E = 16
_K = 2
_NTOK = 16384
_NFLAT = _NTOK * _K

_BM = 1024
_NSUB = 16
_TOK_PER_TILE = _NTOK // _NSUB
_ROWS = 16


def _gate_body(x_ref, w_ref, b_ref, s1_ref, s2_ref, e1_ref, e2_ref, h_ref):
    x = x_ref[...]
    w = w_ref[...]
    b = b_ref[...]
    z = jnp.dot(x, w, preferred_element_type=jnp.float32) + b
    zt = z.T
    rows = lax.broadcasted_iota(jnp.int32, zt.shape, 0)
    m1 = jnp.max(zt, axis=0)
    a1 = jnp.min(jnp.where(zt == m1[None, :], rows, _E), axis=0).astype(jnp.int32)
    z2 = jnp.where(rows == a1[None, :], -jnp.inf, zt)
    m2 = jnp.max(z2, axis=0)
    a2 = jnp.min(jnp.where(z2 == m2[None, :], rows, _E), axis=0).astype(jnp.int32)
    t = jnp.exp(m2 - m1)
    s1 = 1.0 / (1.0 + t)
    s1_ref[...] = s1
    s2_ref[...] = t * s1
    e1_ref[...] = a1
    e2_ref[...] = a2
    onehot = (rows == a1[None, :]).astype(jnp.int32) + (
        rows == a2[None, :]
    ).astype(jnp.int32)
    h_ref[...] = jnp.sum(onehot, axis=1)[None, None, :]


def _gate(x, W, b2d):
    grid = (_NTOK // _BM,)
    return pl.pallas_call(
        _gate_body,
        grid=grid,
        in_specs=[
            pl.BlockSpec((_BM, _DIM), lambda i: (i, 0)),
            pl.BlockSpec((_DIM, _E), lambda i: (0, 0)),
            pl.BlockSpec((1, _E), lambda i: (0, 0)),
        ],
        out_specs=[
            pl.BlockSpec((_BM,), lambda i: (i,)),
            pl.BlockSpec((_BM,), lambda i: (i,)),
            pl.BlockSpec((_BM,), lambda i: (i,)),
            pl.BlockSpec((_BM,), lambda i: (i,)),
            pl.BlockSpec((1, 1, _E), lambda i: (i, 0, 0)),
        ],
        out_shape=[
            jax.ShapeDtypeStruct((_NTOK,), jnp.float32),
            jax.ShapeDtypeStruct((_NTOK,), jnp.float32),
            jax.ShapeDtypeStruct((_NTOK,), jnp.int32),
            jax.ShapeDtypeStruct((_NTOK,), jnp.int32),
            jax.ShapeDtypeStruct((_NTOK // _BM, 1, _E), jnp.int32),
        ],
    )(x, W, b2d)


def _route_body(
    e1_hbm, e2_hbm, s1_hbm, s2_hbm, h_hbm,
    oscore_hbm, otok_hbm, ocnt_hbm,
    e1_v, e2_v, s1_v, s2_v,
    ptr_v, cnt_v, histmat_v,
    pos_v, val_v, tok_v,
    sc_sh, tk_sh, drain_f, drain_i, sem,
):
    wid = lax.axis_index("s")
    tbase = wid * _TOK_PER_TILE
    in_copies = [
        pltpu.async_copy(e1_hbm.at[pl.ds(tbase, _TOK_PER_TILE)], e1_v, sem),
        pltpu.async_copy(e2_hbm.at[pl.ds(tbase, _TOK_PER_TILE)], e2_v, sem),
        pltpu.async_copy(s1_hbm.at[pl.ds(tbase, _TOK_PER_TILE)], s1_v, sem),
        pltpu.async_copy(s2_hbm.at[pl.ds(tbase, _TOK_PER_TILE)], s2_v, sem),
        pltpu.async_copy(h_hbm, histmat_v, sem),
    ]
    for c in in_copies:
        c.wait()

    total = jnp.zeros((_E,), jnp.int32)
    prefix = jnp.zeros((_E,), jnp.int32)
    for w in range(_NSUB):
        h = histmat_v[w, 0]
        total = total + h
        prefix = prefix + h * (jnp.int32(w) < wid).astype(jnp.int32)
    incl = plsc.cumsum(total)
    start = (incl - total) + prefix
    ptr_v[...] = start

    @pl.when(wid == 0)
    def _():
        cnt_v[...] = total
        pltpu.sync_copy(cnt_v, ocnt_hbm)

    iota = lax.iota(jnp.int32, 16)
    half = iota >> 1
    parity = (iota & 1) == 1

    def _row_loop(r, c):
        for h in range(8):
            tok_idx = r * 64 + h * 8 + half
            ea = plsc.load_gather(e1_v, [tok_idx])
            eb = plsc.load_gather(e2_v, [tok_idx])
            ev = jnp.where(parity, eb, ea)
            occ, last = plsc.scan_count(ev)
            base = plsc.load_gather(ptr_v, [ev])
            pos = base + occ - 1
            plsc.store_scatter(ptr_v, [ev], pos + 1, mask=last)
            sa = plsc.load_gather(s1_v, [tok_idx])
            sb = plsc.load_gather(s2_v, [tok_idx])
            sv = jnp.where(parity, sb, sa)
            pos_v[r, pl.ds(h * 16, 16)] = pos
            val_v[r, pl.ds(h * 16, 16)] = sv
            tok_v[r, pl.ds(h * 16, 16)] = tbase + tok_idx
        return c

    lax.fori_loop(0, _ROWS, _row_loop, 0)

    copies = []
    for r in range(_ROWS):
        copies.append(pltpu.async_copy(val_v.at[r], sc_sh.at[pos_v.at[r]], sem))
        copies.append(pltpu.async_copy(tok_v.at[r], tk_sh.at[pos_v.at[r]], sem))
    for c in copies:
        c.wait()
    plsc.subcore_barrier()
    obase = wid * (2 * _TOK_PER_TILE)
    pltpu.sync_copy(sc_sh.at[pl.ds(obase, 2 * _TOK_PER_TILE)], drain_f)
    pltpu.sync_copy(tk_sh.at[pl.ds(obase, 2 * _TOK_PER_TILE)], drain_i)
    pltpu.sync_copy(drain_f, oscore_hbm.at[pl.ds(obase, 2 * _TOK_PER_TILE)])
    pltpu.sync_copy(drain_i, otok_hbm.at[pl.ds(obase, 2 * _TOK_PER_TILE)])


@functools.cache
def _make_route():
    return pl.kernel(
        _route_body,
        out_type=[
            jax.ShapeDtypeStruct((_NFLAT,), jnp.float32),
            jax.ShapeDtypeStruct((_NFLAT,), jnp.int32),
            jax.ShapeDtypeStruct((_E,), jnp.int32),
        ],
        mesh=plsc.VectorSubcoreMesh(
            core_axis_name="c", subcore_axis_name="s", num_cores=1
        ),
        compiler_params=pltpu.CompilerParams(needs_layout_passes=False),
        scratch_types=[
            pltpu.VMEM((_TOK_PER_TILE,), jnp.int32),
            pltpu.VMEM((_TOK_PER_TILE,), jnp.int32),
            pltpu.VMEM((_TOK_PER_TILE,), jnp.float32),
            pltpu.VMEM((_TOK_PER_TILE,), jnp.float32),
            pltpu.VMEM((_E,), jnp.int32),
            pltpu.VMEM((_E,), jnp.int32),
            pltpu.VMEM((_NSUB, 1, _E), jnp.int32),
            pltpu.VMEM((_ROWS, 128), jnp.int32),
            pltpu.VMEM((_ROWS, 128), jnp.float32),
            pltpu.VMEM((_ROWS, 128), jnp.int32),
            pltpu.VMEM_SHARED((_NFLAT,), jnp.float32),
            pltpu.VMEM_SHARED((_NFLAT,), jnp.int32),
            pltpu.VMEM((2 * _TOK_PER_TILE,), jnp.float32),
            pltpu.VMEM((2 * _TOK_PER_TILE,), jnp.int32),
            pltpu.SemaphoreType.DMA,
        ],
    )


def kernel(x, W, b):
    s1, s2, e1, e2, hist = _gate(x, W, b.reshape(1, _E))
    top_scores_sorted, token_indices, num_tokens_per_expert = _make_route()(
        e1, e2, s1, s2, hist
    )
    return (top_scores_sorted, token_indices, num_tokens_per_expert)
